# Initial kernel scaffold; baseline (speedup 1.0000x reference)
import jax
import jax.numpy as jnp
from jax.experimental import pallas as pl


def kernel(indices, embedding):
    raise NotImplementedError("write your pallas kernel here")


# speedup vs baseline: 1.0350x; 1.0350x over previous
import functools

import jax
import jax.numpy as jnp
from jax import lax
from jax.experimental import pallas as pl
from jax.experimental.pallas import tpu as pltpu
from jax.experimental.pallas import tpu_sc as plsc

_D = 1000
_NW = 32
_CHUNK = 40
_NBUF = 2


def _make_gather(n_total):
  per_w = n_total // _NW
  n_chunks = per_w // _CHUNK
  mesh = plsc.VectorSubcoreMesh(core_axis_name="c", subcore_axis_name="s")

  @functools.partial(
      pl.kernel,
      out_type=jax.ShapeDtypeStruct((n_total, _D), jnp.float32),
      mesh=mesh,
      compiler_params=pltpu.CompilerParams(use_tc_tiling_on_sc=False),
      scratch_types=[
          pltpu.VMEM((n_chunks, _CHUNK), jnp.int32),
          pltpu.VMEM((_NBUF, _CHUNK, _D), jnp.float32),
          pltpu.SemaphoreType.DMA,
          pltpu.SemaphoreType.DMA,
          pltpu.SemaphoreType.DMA,
          pltpu.SemaphoreType.DMA,
      ],
  )
  def gather_kernel(idx_hbm, table_hbm, out_hbm, idx_v, rows_v, sg0, sg1,
                    so0, so1):
    wid = lax.axis_index("s") * 2 + lax.axis_index("c")
    base = wid * per_w
    pltpu.sync_copy(idx_hbm.at[wid], idx_v)

    sems_g = (sg0, sg1)
    sems_o = (so0, so1)

    def start_gather(j, b):
      pltpu.async_copy(table_hbm.at[idx_v.at[j]], rows_v.at[b], sems_g[b])

    def wait_gather(j, b):
      pltpu.make_async_copy(table_hbm.at[idx_v.at[j]], rows_v.at[b],
                            sems_g[b]).wait()

    def out_ref(j):
      return out_hbm.at[pl.ds(base + j * _CHUNK, _CHUNK)]

    def start_write(j, b):
      pltpu.async_copy(rows_v.at[b], out_ref(j), sems_o[b])

    def wait_write(j, b):
      pltpu.make_async_copy(rows_v.at[b], out_ref(j), sems_o[b]).wait()

    for b in range(_NBUF):
      start_gather(b, b)

    @pl.loop(0, n_chunks - _NBUF, step=_NBUF)
    def _(j0):
      for b in range(_NBUF):
        j = j0 + b
        wait_gather(j, b)
        start_write(j, b)
        wait_write(j, b)
        start_gather(j + _NBUF, b)

    for b in range(_NBUF):
      j = n_chunks - _NBUF + b
      wait_gather(j, b)
      start_write(j, b)
      wait_write(j, b)

  return gather_kernel


@jax.jit
def kernel(indices, embedding):
  batch, seq = indices.shape
  n_total = batch * seq
  idx = indices.astype(jnp.int32).reshape(_NW, n_total // (_NW * _CHUNK),
                                          _CHUNK)
  out = _make_gather(n_total)(idx, embedding)
  return out.reshape(batch, seq, _D)


# speedup vs baseline: 1.0363x; 1.0012x over previous
import functools

import jax
import jax.numpy as jnp
from jax import lax
from jax.experimental import pallas as pl
from jax.experimental.pallas import tpu as pltpu
from jax.experimental.pallas import tpu_sc as plsc

_D = 1000
_NW = 32
_CHUNK = 16
_NBUF = 4


def _make_gather(n_total):
  per_w = n_total // _NW
  n_chunks = per_w // _CHUNK
  mesh = plsc.VectorSubcoreMesh(core_axis_name="c", subcore_axis_name="s")

  @functools.partial(
      pl.kernel,
      out_type=jax.ShapeDtypeStruct((n_total, _D), jnp.float32),
      mesh=mesh,
      compiler_params=pltpu.CompilerParams(use_tc_tiling_on_sc=False),
      scratch_types=[
          pltpu.VMEM((n_chunks, _CHUNK), jnp.int32),
          pltpu.VMEM((_NBUF, _CHUNK, _D), jnp.float32),
      ] + [pltpu.SemaphoreType.DMA] * (2 * _NBUF),
  )
  def gather_kernel(idx_hbm, table_hbm, out_hbm, idx_v, rows_v, *sems):
    wid = lax.axis_index("s") * 2 + lax.axis_index("c")
    base = wid * per_w
    pltpu.sync_copy(idx_hbm.at[wid], idx_v)

    sems_g = sems[:_NBUF]
    sems_o = sems[_NBUF:]

    def start_gather(j, b):
      pltpu.async_copy(table_hbm.at[idx_v.at[j]], rows_v.at[b], sems_g[b])

    def wait_gather(j, b):
      pltpu.make_async_copy(table_hbm.at[idx_v.at[j]], rows_v.at[b],
                            sems_g[b]).wait()

    def out_ref(j):
      return out_hbm.at[pl.ds(base + j * _CHUNK, _CHUNK)]

    def start_write(j, b):
      pltpu.async_copy(rows_v.at[b], out_ref(j), sems_o[b])

    def wait_write(j, b):
      pltpu.make_async_copy(rows_v.at[b], out_ref(j), sems_o[b]).wait()

    for b in range(_NBUF):
      start_gather(b, b)

    @pl.loop(0, n_chunks - _NBUF, step=_NBUF)
    def _(j0):
      for b in range(_NBUF):
        j = j0 + b
        wait_gather(j, b)
        start_write(j, b)
        wait_write(j, b)
        start_gather(j + _NBUF, b)

    for b in range(_NBUF):
      j = n_chunks - _NBUF + b
      wait_gather(j, b)
      start_write(j, b)
      wait_write(j, b)

  return gather_kernel


@jax.jit
def kernel(indices, embedding):
  batch, seq = indices.shape
  n_total = batch * seq
  idx = indices.astype(jnp.int32).reshape(_NW, n_total // (_NW * _CHUNK),
                                          _CHUNK)
  out = _make_gather(n_total)(idx, embedding)
  return out.reshape(batch, seq, _D)
